# Initial kernel scaffold; baseline (speedup 1.0000x reference)
import functools

import jax
import jax.numpy as jnp
from jax import lax
from jax.experimental import pallas as pl
from jax.experimental.pallas import tpu as pltpu
from jax.experimental.pallas import tpu_sc as plsc

LANES = 128


def _score_body(n, blk, h_ref, wt_ref, b_ref, out_ref):
    i = pl.program_id(1)
    h = h_ref[0]
    w = jnp.sum(h * wt_ref[0][None, :], axis=-1) + b_ref[0, 0]
    gidx = i * blk + lax.broadcasted_iota(jnp.int32, (blk,), 0)
    w = jnp.where(gidx < n, w, -1e30)
    out_ref[0] = w.reshape(blk // LANES, LANES)


def _scores(h, wt, bias, rows):
    b_, n, d = h.shape
    blk = 2048
    nblk = (n + blk - 1) // blk
    return pl.pallas_call(
        functools.partial(_score_body, n, blk),
        grid=(b_, nblk),
        in_specs=[
            pl.BlockSpec((1, blk, d), lambda b, i: (b, i, 0)),
            pl.BlockSpec((1, d), lambda b, i: (0, 0)),
            pl.BlockSpec((1, 1), lambda b, i: (0, 0)),
        ],
        out_specs=pl.BlockSpec((1, blk // LANES, LANES), lambda b, i: (b, i, 0)),
        out_shape=jax.ShapeDtypeStruct((b_, rows, LANES), jnp.float32),
    )(h, wt, bias)


def _cx(key, idx, pos, s, m):
    if s >= LANES:
        axis, amt = 0, s // LANES
    else:
        axis, amt = 1, s
    ku = jnp.roll(key, -amt, axis)
    kd = jnp.roll(key, amt, axis)
    iu = jnp.roll(idx, -amt, axis)
    idn = jnp.roll(idx, amt, axis)
    is_a = (pos & s) == 0
    pk = jnp.where(is_a, ku, kd)
    pi = jnp.where(is_a, iu, idn)
    c = (key > pk) | ((key == pk) & (idx < pi))
    d = (pos & m) == 0
    keep = c == (is_a == d)
    return jnp.where(keep, key, pk), jnp.where(keep, idx, pi)


def _sort_body(n, rows, s_ref, idx_ref, val_ref):
    b = pl.program_id(0)
    row = lax.broadcasted_iota(jnp.int32, (rows, LANES), 0)
    col = lax.broadcasted_iota(jnp.int32, (rows, LANES), 1)
    pos = row * LANES + col
    key = jnp.where(pos < n, s_ref[0], -1.0)
    idx = pos
    total = rows * LANES
    nbits = total.bit_length() - 1
    for j in range(nbits):
        m = 1 << (j + 1)
        for t in range(j, -1, -1):
            key, idx = _cx(key, idx, pos, 1 << t, m)
    idx_ref[0] = idx + b * n
    val_ref[0] = key


def _sort(s3, n):
    b_, rows, _ = s3.shape
    return pl.pallas_call(
        functools.partial(_sort_body, n, rows),
        grid=(b_,),
        in_specs=[pl.BlockSpec((1, rows, LANES), lambda b: (b, 0, 0))],
        out_specs=[
            pl.BlockSpec((1, rows, LANES), lambda b: (b, 0, 0)),
            pl.BlockSpec((1, rows, LANES), lambda b: (b, 0, 0)),
        ],
        out_shape=[
            jax.ShapeDtypeStruct((b_, rows, LANES), jnp.int32),
            jax.ShapeDtypeStruct((b_, rows, LANES), jnp.float32),
        ],
    )(s3)


def _gather_body(nw, chunk, nfull, tail_a, tail_b, d,
                 idx_hbm, h_hbm, out_hbm, idx_v, idx_ta, idx_tb, rows_v, sem):
    wid = lax.axis_index("s") * 2 + lax.axis_index("c")
    base = wid * (nfull * chunk + tail_a)
    for c in range(nfull):
        off = base + c * chunk
        pltpu.sync_copy(idx_hbm.at[pl.ds(off, chunk)], idx_v)
        pltpu.async_copy(h_hbm.at[idx_v], rows_v, sem).wait()
        pltpu.sync_copy(rows_v, out_hbm.at[pl.ds(off, chunk)])
    off = base + nfull * chunk

    @pl.when(wid < nw - 1)
    def _():
        pltpu.sync_copy(idx_hbm.at[pl.ds(off, tail_a)], idx_ta)
        pltpu.async_copy(h_hbm.at[idx_ta], rows_v.at[pl.ds(0, tail_a)], sem).wait()
        pltpu.sync_copy(rows_v.at[pl.ds(0, tail_a)], out_hbm.at[pl.ds(off, tail_a)])

    @pl.when(wid == nw - 1)
    def _():
        pltpu.sync_copy(idx_hbm.at[pl.ds(off, tail_b)], idx_tb)
        pltpu.async_copy(h_hbm.at[idx_tb], rows_v.at[pl.ds(0, tail_b)], sem).wait()
        pltpu.sync_copy(rows_v.at[pl.ds(0, tail_b)], out_hbm.at[pl.ds(off, tail_b)])


def _gather(idx_flat, h2d):
    total = idx_flat.shape[0]
    d = h2d.shape[1]
    info = plsc.get_sparse_core_info()
    nw = info.num_cores * info.num_subcores
    chunk = 392
    per_w = -(-total // nw)
    per_w = ((per_w + 7) // 8) * 8
    nfull = per_w // chunk
    tail_a = per_w - nfull * chunk
    tail_b = total - (nw - 1) * per_w - nfull * chunk
    assert tail_a > 0 and tail_b > 0 and tail_a % 8 == 0 and tail_b % 8 == 0
    mesh = plsc.VectorSubcoreMesh(core_axis_name="c", subcore_axis_name="s")
    fn = pl.kernel(
        functools.partial(_gather_body, nw, chunk, nfull, tail_a, tail_b, d),
        out_type=jax.ShapeDtypeStruct((total, d), jnp.float32),
        mesh=mesh,
        scratch_types=[
            pltpu.VMEM((chunk,), jnp.int32),
            pltpu.VMEM((tail_a,), jnp.int32),
            pltpu.VMEM((tail_b,), jnp.int32),
            pltpu.VMEM((chunk, d), jnp.float32),
            pltpu.SemaphoreType.DMA,
        ],
    )
    return fn(idx_flat, h2d)


def _scale_body(g_ref, s_ref, o_ref):
    o_ref[...] = g_ref[...] * s_ref[...]


def _scale(g, sv):
    total, d = g.shape
    blk = 2500
    return pl.pallas_call(
        _scale_body,
        grid=(total // blk,),
        in_specs=[
            pl.BlockSpec((blk, d), lambda i: (i, 0)),
            pl.BlockSpec((blk, 1), lambda i: (i, 0)),
        ],
        out_specs=pl.BlockSpec((blk, d), lambda i: (i, 0)),
        out_shape=jax.ShapeDtypeStruct((total, d), jnp.float32),
    )(g, sv)


def kernel(h, W, b):
    b_, n, d = h.shape
    k = max(int(n * 0.5), 1)
    total = 1 << ((n - 1).bit_length())
    rows = total // LANES
    wt = W.reshape(1, d)
    bias = b.reshape(1, 1)

    w3 = _scores(h, wt, bias, rows)
    s3 = jax.nn.sigmoid(w3)
    idx3, val3 = _sort(s3, n)
    idx_flat = idx3.reshape(b_, total)[:, :k].reshape(b_ * k)
    vals = val3.reshape(b_, total)[:, :k].reshape(b_ * k, 1)
    g = _gather(idx_flat, h.reshape(b_ * n, d))
    out = _scale(g, vals)
    return out.reshape(b_, k, d)


# speedup vs baseline: 1.2466x; 1.2466x over previous
import functools

import jax
import jax.numpy as jnp
from jax import lax
from jax.experimental import pallas as pl
from jax.experimental.pallas import tpu as pltpu
from jax.experimental.pallas import tpu_sc as plsc

LANES = 128


def _score_body(n, blk, h_ref, wt_ref, b_ref, out_ref):
    i = pl.program_id(1)
    hb = h_ref[0].astype(jnp.bfloat16)
    wb = wt_ref[...].astype(jnp.bfloat16)
    w = lax.dot_general(hb, wb, (((1,), (0,)), ((), ())),
                        preferred_element_type=jnp.float32)
    w = w + b_ref[0, 0]
    gidx = i * blk + lax.broadcasted_iota(jnp.int32, (blk, 1), 0)
    w = jnp.where(gidx < n, w, -1e30)
    out_ref[0] = w.reshape(blk // LANES, LANES)


def _scores(h, wt, bias, rows):
    b_, n, d = h.shape
    blk = 2048
    nblk = (n + blk - 1) // blk
    return pl.pallas_call(
        functools.partial(_score_body, n, blk),
        grid=(b_, nblk),
        in_specs=[
            pl.BlockSpec((1, blk, d), lambda b, i: (b, i, 0)),
            pl.BlockSpec((d, 1), lambda b, i: (0, 0)),
            pl.BlockSpec((1, 1), lambda b, i: (0, 0)),
        ],
        out_specs=pl.BlockSpec((1, blk // LANES, LANES), lambda b, i: (b, i, 0)),
        out_shape=jax.ShapeDtypeStruct((b_, rows, LANES), jnp.float32),
    )(h, wt, bias)


def _cx(key, idx, pos, s, m):
    if s >= LANES:
        axis, amt = 0, s // LANES
    else:
        axis, amt = 1, s
    ku = jnp.roll(key, -amt, axis)
    kd = jnp.roll(key, amt, axis)
    iu = jnp.roll(idx, -amt, axis)
    idn = jnp.roll(idx, amt, axis)
    is_a = (pos & s) == 0
    pk = jnp.where(is_a, ku, kd)
    pi = jnp.where(is_a, iu, idn)
    c = (key > pk) | ((key == pk) & (idx < pi))
    d = (pos & m) == 0
    keep = c == (is_a == d)
    return jnp.where(keep, key, pk), jnp.where(keep, idx, pi)


def _sort_body(n, rows, s_ref, idx_ref, val_ref):
    b = pl.program_id(0)
    row = lax.broadcasted_iota(jnp.int32, (rows, LANES), 0)
    col = lax.broadcasted_iota(jnp.int32, (rows, LANES), 1)
    pos = row * LANES + col
    key = jnp.where(pos < n, s_ref[0], -1.0)
    idx = pos
    total = rows * LANES
    nbits = total.bit_length() - 1
    for j in range(nbits):
        m = 1 << (j + 1)
        for t in range(j, -1, -1):
            key, idx = _cx(key, idx, pos, 1 << t, m)
    idx_ref[0] = idx + b * n
    val_ref[0] = key


def _sort(s3, n):
    b_, rows, _ = s3.shape
    return pl.pallas_call(
        functools.partial(_sort_body, n, rows),
        grid=(b_,),
        in_specs=[pl.BlockSpec((1, rows, LANES), lambda b: (b, 0, 0))],
        out_specs=[
            pl.BlockSpec((1, rows, LANES), lambda b: (b, 0, 0)),
            pl.BlockSpec((1, rows, LANES), lambda b: (b, 0, 0)),
        ],
        out_shape=[
            jax.ShapeDtypeStruct((b_, rows, LANES), jnp.int32),
            jax.ShapeDtypeStruct((b_, rows, LANES), jnp.float32),
        ],
    )(s3)


def _gather_body(nw, chunk, nfull, tail_a, tail_b, d,
                 idx_hbm, h_hbm, out_hbm, idx_v, idx_ta, idx_tb, rows_v, sem):
    wid = lax.axis_index("s") * 2 + lax.axis_index("c")
    base = wid * (nfull * chunk + tail_a)
    for c in range(nfull):
        off = base + c * chunk
        pltpu.sync_copy(idx_hbm.at[pl.ds(off, chunk)], idx_v)
        pltpu.async_copy(h_hbm.at[idx_v], rows_v, sem).wait()
        pltpu.sync_copy(rows_v, out_hbm.at[pl.ds(off, chunk)])
    off = base + nfull * chunk

    @pl.when(wid < nw - 1)
    def _():
        pltpu.sync_copy(idx_hbm.at[pl.ds(off, tail_a)], idx_ta)
        pltpu.async_copy(h_hbm.at[idx_ta], rows_v.at[pl.ds(0, tail_a)], sem).wait()
        pltpu.sync_copy(rows_v.at[pl.ds(0, tail_a)], out_hbm.at[pl.ds(off, tail_a)])

    @pl.when(wid == nw - 1)
    def _():
        pltpu.sync_copy(idx_hbm.at[pl.ds(off, tail_b)], idx_tb)
        pltpu.async_copy(h_hbm.at[idx_tb], rows_v.at[pl.ds(0, tail_b)], sem).wait()
        pltpu.sync_copy(rows_v.at[pl.ds(0, tail_b)], out_hbm.at[pl.ds(off, tail_b)])


def _gather(idx_flat, h2d):
    total = idx_flat.shape[0]
    d = h2d.shape[1]
    info = plsc.get_sparse_core_info()
    nw = info.num_cores * info.num_subcores
    chunk = 392
    per_w = -(-total // nw)
    per_w = ((per_w + 7) // 8) * 8
    nfull = per_w // chunk
    tail_a = per_w - nfull * chunk
    tail_b = total - (nw - 1) * per_w - nfull * chunk
    assert tail_a > 0 and tail_b > 0 and tail_a % 8 == 0 and tail_b % 8 == 0
    mesh = plsc.VectorSubcoreMesh(core_axis_name="c", subcore_axis_name="s")
    fn = pl.kernel(
        functools.partial(_gather_body, nw, chunk, nfull, tail_a, tail_b, d),
        out_type=jax.ShapeDtypeStruct((total, d), jnp.float32),
        mesh=mesh,
        scratch_types=[
            pltpu.VMEM((chunk,), jnp.int32),
            pltpu.VMEM((tail_a,), jnp.int32),
            pltpu.VMEM((tail_b,), jnp.int32),
            pltpu.VMEM((chunk, d), jnp.float32),
            pltpu.SemaphoreType.DMA,
        ],
    )
    return fn(idx_flat, h2d)


def _scale_body(g_ref, s_ref, o_ref):
    o_ref[...] = g_ref[...] * s_ref[...]


def _scale(g, sv):
    total, d = g.shape
    blk = 2000
    return pl.pallas_call(
        _scale_body,
        grid=(total // blk,),
        in_specs=[
            pl.BlockSpec((blk, d), lambda i: (i, 0)),
            pl.BlockSpec((blk, 1), lambda i: (i, 0)),
        ],
        out_specs=pl.BlockSpec((blk, d), lambda i: (i, 0)),
        out_shape=jax.ShapeDtypeStruct((total, d), jnp.float32),
    )(g, sv)


def kernel(h, W, b):
    b_, n, d = h.shape
    k = max(int(n * 0.5), 1)
    total = 1 << ((n - 1).bit_length())
    rows = total // LANES
    wt = W
    bias = b.reshape(1, 1)

    w3 = _scores(h, wt, bias, rows)
    s3 = jax.nn.sigmoid(w3)
    idx3, val3 = _sort(s3, n)
    idx_flat = idx3.reshape(b_, total)[:, :k].reshape(b_ * k)
    vals = val3.reshape(b_, total)[:, :k].reshape(b_ * k, 1)
    g = _gather(idx_flat, h.reshape(b_ * n, d))
    out = _scale(g, vals)
    return out.reshape(b_, k, d)


# speedup vs baseline: 1.3034x; 1.0456x over previous
import functools

import jax
import jax.numpy as jnp
from jax import lax
from jax.experimental import pallas as pl
from jax.experimental.pallas import tpu as pltpu
from jax.experimental.pallas import tpu_sc as plsc

LANES = 128


def _score_body(n, blk, h_ref, wt_ref, b_ref, out_ref):
    i = pl.program_id(1)
    hb = h_ref[0].astype(jnp.bfloat16)
    wb = wt_ref[...].astype(jnp.bfloat16)
    w = lax.dot_general(hb, wb, (((1,), (0,)), ((), ())),
                        preferred_element_type=jnp.float32)
    w = w + b_ref[0, 0]
    gidx = i * blk + lax.broadcasted_iota(jnp.int32, (blk, 1), 0)
    w = jnp.where(gidx < n, w, -1e30)
    out_ref[0] = w.reshape(blk // LANES, LANES)


def _scores(h, wt, bias, rows):
    b_, n, d = h.shape
    blk = 2048
    nblk = (n + blk - 1) // blk
    return pl.pallas_call(
        functools.partial(_score_body, n, blk),
        grid=(b_, nblk),
        in_specs=[
            pl.BlockSpec((1, blk, d), lambda b, i: (b, i, 0)),
            pl.BlockSpec((d, 1), lambda b, i: (0, 0)),
            pl.BlockSpec((1, 1), lambda b, i: (0, 0)),
        ],
        out_specs=pl.BlockSpec((1, blk // LANES, LANES), lambda b, i: (b, i, 0)),
        out_shape=jax.ShapeDtypeStruct((b_, rows, LANES), jnp.float32),
    )(h, wt, bias)


def _cx(key, idx, pos, s, m):
    if s >= LANES:
        axis, amt, size = 0, s // LANES, key.shape[0]
    else:
        axis, amt, size = 1, s, LANES
    ku = pltpu.roll(key, size - amt, axis)
    kd = pltpu.roll(key, amt, axis)
    iu = pltpu.roll(idx, size - amt, axis)
    idn = pltpu.roll(idx, amt, axis)
    is_a = (pos & s) == 0
    pk = jnp.where(is_a, ku, kd)
    pi = jnp.where(is_a, iu, idn)
    c = (key > pk) | ((key == pk) & (idx < pi))
    d = (pos & m) == 0
    keep = c == (is_a == d)
    return jnp.where(keep, key, pk), jnp.where(keep, idx, pi)


def _cx_rows(key, idx, s, m):
    amt = s // LANES
    r = key.shape[0]
    ks, vs = [], []
    for i0 in range(0, r, 2 * amt):
        ak, bk = key[i0:i0 + amt], key[i0 + amt:i0 + 2 * amt]
        ai, bi = idx[i0:i0 + amt], idx[i0 + amt:i0 + 2 * amt]
        c = (ak > bk) | ((ak == bk) & (ai < bi))
        if ((i0 * LANES) & m) == 0:
            ks += [jnp.where(c, ak, bk), jnp.where(c, bk, ak)]
            vs += [jnp.where(c, ai, bi), jnp.where(c, bi, ai)]
        else:
            ks += [jnp.where(c, bk, ak), jnp.where(c, ak, bk)]
            vs += [jnp.where(c, bi, ai), jnp.where(c, ai, bi)]
    return jnp.concatenate(ks, 0), jnp.concatenate(vs, 0)


def _substage(key, idx, pos, s, m):
    if s >= 8 * LANES:
        return _cx_rows(key, idx, s, m)
    return _cx(key, idx, pos, s, m)


def _sort_body(n, rows, s_ref, idx_ref, val_ref):
    b = pl.program_id(0)
    row = lax.broadcasted_iota(jnp.int32, (rows, LANES), 0)
    col = lax.broadcasted_iota(jnp.int32, (rows, LANES), 1)
    pos = row * LANES + col
    key = jnp.where(pos < n, s_ref[0], -1.0)
    idx = pos
    total = rows * LANES
    nbits = total.bit_length() - 1
    for j in range(nbits - 1):
        m = 1 << (j + 1)
        for t in range(j, -1, -1):
            key, idx = _substage(key, idx, pos, 1 << t, m)
    m = 1 << nbits
    key, idx = _substage(key, idx, pos, 1 << (nbits - 1), m)
    half = rows // 2
    key, idx, pos = key[:half], idx[:half], pos[:half]
    for t in range(nbits - 2, -1, -1):
        key, idx = _substage(key, idx, pos, 1 << t, m)
    idx_ref[0, pl.ds(0, half)] = idx + b * n
    val_ref[0, pl.ds(0, half)] = key


def _sort(s3, n):
    b_, rows, _ = s3.shape
    return pl.pallas_call(
        functools.partial(_sort_body, n, rows),
        grid=(b_,),
        in_specs=[pl.BlockSpec((1, rows, LANES), lambda b: (b, 0, 0))],
        out_specs=[
            pl.BlockSpec((1, rows, LANES), lambda b: (b, 0, 0)),
            pl.BlockSpec((1, rows, LANES), lambda b: (b, 0, 0)),
        ],
        out_shape=[
            jax.ShapeDtypeStruct((b_, rows, LANES), jnp.int32),
            jax.ShapeDtypeStruct((b_, rows, LANES), jnp.float32),
        ],
    )(s3)


def _gather_body(nw, chunk, nfull, tail_a, tail_b, d,
                 idx_hbm, h_hbm, out_hbm, idx_v, idx_ta, idx_tb, rows_v, sem):
    wid = lax.axis_index("s") * 2 + lax.axis_index("c")
    base = wid * (nfull * chunk + tail_a)
    for c in range(nfull):
        off = base + c * chunk
        pltpu.sync_copy(idx_hbm.at[pl.ds(off, chunk)], idx_v)
        pltpu.async_copy(h_hbm.at[idx_v], rows_v, sem).wait()
        pltpu.sync_copy(rows_v, out_hbm.at[pl.ds(off, chunk)])
    off = base + nfull * chunk

    @pl.when(wid < nw - 1)
    def _():
        pltpu.sync_copy(idx_hbm.at[pl.ds(off, tail_a)], idx_ta)
        pltpu.async_copy(h_hbm.at[idx_ta], rows_v.at[pl.ds(0, tail_a)], sem).wait()
        pltpu.sync_copy(rows_v.at[pl.ds(0, tail_a)], out_hbm.at[pl.ds(off, tail_a)])

    @pl.when(wid == nw - 1)
    def _():
        pltpu.sync_copy(idx_hbm.at[pl.ds(off, tail_b)], idx_tb)
        pltpu.async_copy(h_hbm.at[idx_tb], rows_v.at[pl.ds(0, tail_b)], sem).wait()
        pltpu.sync_copy(rows_v.at[pl.ds(0, tail_b)], out_hbm.at[pl.ds(off, tail_b)])


def _gather(idx_flat, h2d):
    total = idx_flat.shape[0]
    d = h2d.shape[1]
    info = plsc.get_sparse_core_info()
    nw = info.num_cores * info.num_subcores
    chunk = 392
    per_w = -(-total // nw)
    per_w = ((per_w + 7) // 8) * 8
    nfull = per_w // chunk
    tail_a = per_w - nfull * chunk
    tail_b = total - (nw - 1) * per_w - nfull * chunk
    assert tail_a > 0 and tail_b > 0 and tail_a % 8 == 0 and tail_b % 8 == 0
    mesh = plsc.VectorSubcoreMesh(core_axis_name="c", subcore_axis_name="s")
    fn = pl.kernel(
        functools.partial(_gather_body, nw, chunk, nfull, tail_a, tail_b, d),
        out_type=jax.ShapeDtypeStruct((total, d), jnp.float32),
        mesh=mesh,
        scratch_types=[
            pltpu.VMEM((chunk,), jnp.int32),
            pltpu.VMEM((tail_a,), jnp.int32),
            pltpu.VMEM((tail_b,), jnp.int32),
            pltpu.VMEM((chunk, d), jnp.float32),
            pltpu.SemaphoreType.DMA,
        ],
    )
    return fn(idx_flat, h2d)


def _scale_body(g_ref, s_ref, o_ref):
    o_ref[...] = g_ref[...] * s_ref[...]


def _scale(g, sv):
    total, d = g.shape
    blk = 2000
    return pl.pallas_call(
        _scale_body,
        grid=(total // blk,),
        in_specs=[
            pl.BlockSpec((blk, d), lambda i: (i, 0)),
            pl.BlockSpec((blk, 1), lambda i: (i, 0)),
        ],
        out_specs=pl.BlockSpec((blk, d), lambda i: (i, 0)),
        out_shape=jax.ShapeDtypeStruct((total, d), jnp.float32),
    )(g, sv)


def kernel(h, W, b):
    b_, n, d = h.shape
    k = max(int(n * 0.5), 1)
    total = 1 << ((n - 1).bit_length())
    rows = total // LANES
    wt = W
    bias = b.reshape(1, 1)

    w3 = _scores(h, wt, bias, rows)
    s3 = jax.nn.sigmoid(w3)
    idx3, val3 = _sort(s3, n)
    idx_flat = idx3.reshape(b_, total)[:, :k].reshape(b_ * k)
    vals = val3.reshape(b_, total)[:, :k].reshape(b_ * k, 1)
    g = _gather(idx_flat, h.reshape(b_ * n, d))
    out = _scale(g, vals)
    return out.reshape(b_, k, d)


# speedup vs baseline: 1.5178x; 1.1645x over previous
import functools

import jax
import jax.numpy as jnp
from jax import lax
from jax.experimental import pallas as pl
from jax.experimental.pallas import tpu as pltpu
from jax.experimental.pallas import tpu_sc as plsc

LANES = 128


def _score_body(n, blk, h_ref, wt_ref, b_ref, out_ref):
    i = pl.program_id(1)
    hb = h_ref[0].astype(jnp.bfloat16)
    wb = wt_ref[...].astype(jnp.bfloat16)
    w = lax.dot_general(hb, wb, (((1,), (0,)), ((), ())),
                        preferred_element_type=jnp.float32)
    w = w + b_ref[0, 0]
    gidx = i * blk + lax.broadcasted_iota(jnp.int32, (blk, 1), 0)
    w = jnp.where(gidx < n, w, -1e30)
    out_ref[0] = w.reshape(blk // LANES, LANES)


def _scores(h, wt, bias, rows):
    b_, n, d = h.shape
    blk = 2048
    nblk = (n + blk - 1) // blk
    return pl.pallas_call(
        functools.partial(_score_body, n, blk),
        grid=(b_, nblk),
        in_specs=[
            pl.BlockSpec((1, blk, d), lambda b, i: (b, i, 0)),
            pl.BlockSpec((d, 1), lambda b, i: (0, 0)),
            pl.BlockSpec((1, 1), lambda b, i: (0, 0)),
        ],
        out_specs=pl.BlockSpec((1, blk // LANES, LANES), lambda b, i: (b, i, 0)),
        out_shape=jax.ShapeDtypeStruct((b_, rows, LANES), jnp.float32),
    )(h, wt, bias)


def _cx(key, idx, pos, s, m):
    if s >= LANES:
        axis, amt, size = 0, s // LANES, key.shape[0]
    else:
        axis, amt, size = 1, s, LANES
    ku = pltpu.roll(key, size - amt, axis)
    kd = pltpu.roll(key, amt, axis)
    iu = pltpu.roll(idx, size - amt, axis)
    idn = pltpu.roll(idx, amt, axis)
    is_a = (pos & s) == 0
    pk = jnp.where(is_a, ku, kd)
    pi = jnp.where(is_a, iu, idn)
    c = (key > pk) | ((key == pk) & (idx < pi))
    d = (pos & m) == 0
    keep = c == (is_a == d)
    return jnp.where(keep, key, pk), jnp.where(keep, idx, pi)


def _cx_rows(key, idx, s, m):
    amt = s // LANES
    r = key.shape[0]
    ks, vs = [], []
    for i0 in range(0, r, 2 * amt):
        ak, bk = key[i0:i0 + amt], key[i0 + amt:i0 + 2 * amt]
        ai, bi = idx[i0:i0 + amt], idx[i0 + amt:i0 + 2 * amt]
        c = (ak > bk) | ((ak == bk) & (ai < bi))
        if ((i0 * LANES) & m) == 0:
            ks += [jnp.where(c, ak, bk), jnp.where(c, bk, ak)]
            vs += [jnp.where(c, ai, bi), jnp.where(c, bi, ai)]
        else:
            ks += [jnp.where(c, bk, ak), jnp.where(c, ak, bk)]
            vs += [jnp.where(c, bi, ai), jnp.where(c, ai, bi)]
    return jnp.concatenate(ks, 0), jnp.concatenate(vs, 0)


def _substage(key, idx, pos, s, m):
    if s >= 8 * LANES:
        return _cx_rows(key, idx, s, m)
    return _cx(key, idx, pos, s, m)


def _sort_body(n, rows, s_ref, idx_ref, val_ref):
    b = pl.program_id(0)
    row = lax.broadcasted_iota(jnp.int32, (rows, LANES), 0)
    col = lax.broadcasted_iota(jnp.int32, (rows, LANES), 1)
    pos = row * LANES + col
    key = jnp.where(pos < n, s_ref[0], -1.0)
    idx = pos
    total = rows * LANES
    nbits = total.bit_length() - 1
    for j in range(nbits - 1):
        m = 1 << (j + 1)
        for t in range(j, -1, -1):
            key, idx = _substage(key, idx, pos, 1 << t, m)
    m = 1 << nbits
    key, idx = _substage(key, idx, pos, 1 << (nbits - 1), m)
    half = rows // 2
    key, idx, pos = key[:half], idx[:half], pos[:half]
    for t in range(nbits - 2, -1, -1):
        key, idx = _substage(key, idx, pos, 1 << t, m)
    idx_ref[0, pl.ds(0, half)] = idx + b * n
    val_ref[0, pl.ds(0, half)] = key


def _sort(s3, n):
    b_, rows, _ = s3.shape
    return pl.pallas_call(
        functools.partial(_sort_body, n, rows),
        grid=(b_,),
        in_specs=[pl.BlockSpec((1, rows, LANES), lambda b: (b, 0, 0))],
        out_specs=[
            pl.BlockSpec((1, rows, LANES), lambda b: (b, 0, 0)),
            pl.BlockSpec((1, rows, LANES), lambda b: (b, 0, 0)),
        ],
        out_shape=[
            jax.ShapeDtypeStruct((b_, rows, LANES), jnp.int32),
            jax.ShapeDtypeStruct((b_, rows, LANES), jnp.float32),
        ],
    )(s3)


def _mul_rows(rows_ref, sv_ref, off, count, d):
    zero16 = jnp.zeros((16, 1), jnp.int32)
    dn = lax.GatherDimensionNumbers(
        offset_dims=(), collapsed_slice_dims=(0,), start_index_map=(0,))

    def body(j, carry):
        sv16 = sv_ref[pl.ds(off + j, 16)]
        sp = lax.gather(sv16, zero16, dn, slice_sizes=(1,),
                        mode=lax.GatherScatterMode.PROMISE_IN_BOUNDS)
        for v in range(d // 16):
            rows_ref[j, pl.ds(v * 16, 16)] = rows_ref[j, pl.ds(v * 16, 16)] * sp
        return carry

    lax.fori_loop(0, count, body, 0, unroll=False)


def _gather_body(nw, chunk, nfull, per_w, tail_a, tail_b, d,
                 idx_hbm, sv_hbm, h_hbm, out_hbm,
                 idx_v, sv_v, rows0, rows1, sem0, sem1):
    wid = lax.axis_index("s") * 2 + lax.axis_index("c")
    base = wid * per_w

    @pl.when(wid < nw - 1)
    def _():
        pltpu.sync_copy(idx_hbm.at[pl.ds(base, per_w)], idx_v)
        pltpu.sync_copy(sv_hbm.at[pl.ds(base, per_w)], sv_v.at[pl.ds(0, per_w)])

    @pl.when(wid == nw - 1)
    def _():
        lastw = nfull * chunk + tail_b
        pltpu.sync_copy(idx_hbm.at[pl.ds(base, lastw)], idx_v.at[pl.ds(0, lastw)])
        pltpu.sync_copy(sv_hbm.at[pl.ds(base, lastw)], sv_v.at[pl.ds(0, lastw)])

    bufs = (rows0, rows1)
    sems = (sem0, sem1)
    handles = [None, None]
    for c in range(nfull):
        handles[c % 2] = pltpu.async_copy(
            h_hbm.at[idx_v.at[pl.ds(c * chunk, chunk)]], bufs[c % 2], sems[c % 2])
        if c > 0:
            p = c - 1
            handles[p % 2].wait()
            _mul_rows(bufs[p % 2], sv_v, p * chunk, chunk, d)
            pltpu.sync_copy(bufs[p % 2], out_hbm.at[pl.ds(base + p * chunk, chunk)])
    p = nfull - 1
    handles[p % 2].wait()
    _mul_rows(bufs[p % 2], sv_v, p * chunk, chunk, d)
    pltpu.sync_copy(bufs[p % 2], out_hbm.at[pl.ds(base + p * chunk, chunk)])

    off = nfull * chunk

    @pl.when(wid < nw - 1)
    def _():
        pltpu.async_copy(h_hbm.at[idx_v.at[pl.ds(off, tail_a)]],
                         rows0.at[pl.ds(0, tail_a)], sem0).wait()
        _mul_rows(rows0, sv_v, off, tail_a, d)
        pltpu.sync_copy(rows0.at[pl.ds(0, tail_a)],
                        out_hbm.at[pl.ds(base + off, tail_a)])

    @pl.when(wid == nw - 1)
    def _():
        pltpu.async_copy(h_hbm.at[idx_v.at[pl.ds(off, tail_b)]],
                         rows0.at[pl.ds(0, tail_b)], sem0).wait()
        _mul_rows(rows0, sv_v, off, tail_b, d)
        pltpu.sync_copy(rows0.at[pl.ds(0, tail_b)],
                        out_hbm.at[pl.ds(base + off, tail_b)])


def _gather(idx_flat, sv_flat, h2d):
    total = idx_flat.shape[0]
    d = h2d.shape[1]
    info = plsc.get_sparse_core_info()
    nw = info.num_cores * info.num_subcores
    chunk = 392
    per_w = -(-total // nw)
    per_w = ((per_w + 7) // 8) * 8
    nfull = per_w // chunk
    tail_a = per_w - nfull * chunk
    tail_b = total - (nw - 1) * per_w - nfull * chunk
    assert tail_a > 0 and tail_b > 0 and tail_a % 8 == 0 and tail_b % 8 == 0
    mesh = plsc.VectorSubcoreMesh(core_axis_name="c", subcore_axis_name="s")
    fn = pl.kernel(
        functools.partial(_gather_body, nw, chunk, nfull, per_w, tail_a, tail_b, d),
        out_type=jax.ShapeDtypeStruct((total, d), jnp.float32),
        mesh=mesh,
        scratch_types=[
            pltpu.VMEM((per_w,), jnp.int32),
            pltpu.VMEM((per_w + 16,), jnp.float32),
            pltpu.VMEM((chunk, d), jnp.float32),
            pltpu.VMEM((chunk, d), jnp.float32),
            pltpu.SemaphoreType.DMA,
            pltpu.SemaphoreType.DMA,
        ],
    )
    return fn(idx_flat, sv_flat, h2d)


def kernel(h, W, b):
    b_, n, d = h.shape
    k = max(int(n * 0.5), 1)
    total = 1 << ((n - 1).bit_length())
    rows = total // LANES
    wt = W
    bias = b.reshape(1, 1)

    w3 = _scores(h, wt, bias, rows)
    s3 = jax.nn.sigmoid(w3)
    idx3, val3 = _sort(s3, n)
    idx_flat = idx3.reshape(b_, total)[:, :k].reshape(b_ * k)
    vals = val3.reshape(b_, total)[:, :k].reshape(b_ * k)
    out = _gather(idx_flat, vals, h.reshape(b_ * n, d))
    return out.reshape(b_, k, d)
